# Initial kernel scaffold; baseline (speedup 1.0000x reference)
import jax
import jax.numpy as jnp
from jax.experimental import pallas as pl


def kernel(context, A_tables, C_last):
    raise NotImplementedError("write your pallas kernel here")


# speedup vs baseline: 27.2085x; 27.2085x over previous
import functools

import jax
import jax.numpy as jnp
from jax import lax
from jax.experimental import pallas as pl
from jax.experimental.pallas import tpu as pltpu
from jax.experimental.pallas import tpu_sc as plsc

L = 16


def _seg_sum_body(ntab, seg_per_worker, G, S, E, *refs):
    ctx_hbm = refs[0]
    tabs = refs[1:1 + ntab]
    out_hbm = refs[1 + ntab]
    idx_v, rows_v, acc_v, sem = refs[2 + ntab:]

    nc = plsc.get_sparse_core_info().num_cores
    wid = lax.axis_index("s") * nc + lax.axis_index("c")
    seg0 = wid * seg_per_worker
    nch = seg_per_worker // G

    def chunk_body(c, carry):
        sbase = seg0 + c * G
        pltpu.sync_copy(ctx_hbm.at[pl.ds(sbase * S, G * S)], idx_v)
        for t, tab in enumerate(tabs):
            pltpu.async_copy(tab.at[idx_v], rows_v, sem).wait()

            def seg_body(g, carry2):
                a0 = jnp.zeros((L,), jnp.float32)
                a1 = jnp.zeros((L,), jnp.float32)
                r0 = g * S
                for s in range(S):
                    a0 = a0 + rows_v[r0 + s, pl.ds(0, L)]
                    a1 = a1 + rows_v[r0 + s, pl.ds(L, L)]
                acc_v[g, pl.ds(0, L)] = a0
                acc_v[g, pl.ds(L, L)] = a1
                return carry2

            lax.fori_loop(0, G, seg_body, 0)
            pltpu.sync_copy(acc_v, out_hbm.at[t, pl.ds(sbase, G)])
        return carry

    lax.fori_loop(0, nch, chunk_body, 0)


def _sc_segment_sums(ctx, tables, nsegs, S, E):
    ntab = len(tables)
    info = plsc.get_sparse_core_info()
    nworkers = info.num_cores * info.num_subcores
    seg_per_worker = nsegs // nworkers
    G = 40
    assert seg_per_worker % G == 0 and G % 8 == 0 and (G * S) % 8 == 0

    mesh = plsc.VectorSubcoreMesh(core_axis_name="c", subcore_axis_name="s")
    kfn = pl.kernel(
        functools.partial(_seg_sum_body, ntab, seg_per_worker, G, S, E),
        out_type=jax.ShapeDtypeStruct((ntab, nsegs, E), jnp.float32),
        mesh=mesh,
        scratch_types=[
            pltpu.VMEM((G * S,), jnp.int32),
            pltpu.VMEM((G * S, E), jnp.float32),
            pltpu.VMEM((G, E), jnp.float32),
            pltpu.SemaphoreType.DMA,
        ],
        compiler_params=pltpu.CompilerParams(use_tc_tiling_on_sc=False),
    )
    return kfn(ctx, *tables)


def _chain_body(E, sa1_ref, sa2_ref, scl_ref, o_ref):
    sa1 = sa1_ref[...]
    sa2 = sa2_ref[...]
    scl = scl_ref[...]
    q = sa1 * (1.0 / E)
    attn = jax.nn.softmax(sa1 * q, axis=-1)
    q = q + sa2 * attn
    attn = jax.nn.softmax(sa2 * q, axis=-1)
    o_ref[...] = scl * attn


def _tc_chain(sa1, sa2, scl):
    nsegs, E = sa1.shape
    BLK = 2048
    grid = (nsegs // BLK,)
    spec = pl.BlockSpec((BLK, E), lambda i: (i, 0))
    return pl.pallas_call(
        functools.partial(_chain_body, E),
        grid=grid,
        in_specs=[spec, spec, spec],
        out_specs=spec,
        out_shape=jax.ShapeDtypeStruct((nsegs, E), jnp.float32),
    )(sa1, sa2, scl)


def kernel(context, A_tables, C_last):
    B, M, S = context.shape
    E = A_tables.shape[-1]
    nsegs = B * M
    ctx = context.reshape(-1)
    tables = (A_tables[1], A_tables[2], C_last)
    sums = _sc_segment_sums(ctx, tables, nsegs, S, E)
    out = _tc_chain(sums[0], sums[1], sums[2])
    return out.reshape(B, M, E)


# speedup vs baseline: 33.8342x; 1.2435x over previous
import functools

import jax
import jax.numpy as jnp
from jax import lax
from jax.experimental import pallas as pl
from jax.experimental.pallas import tpu as pltpu
from jax.experimental.pallas import tpu_sc as plsc

L = 16


def _seg_sum_body(ntab, seg_per_worker, G, S, E, *refs):
    ctx_hbm = refs[0]
    tabs = refs[1:1 + ntab]
    out_hbm = refs[1 + ntab]
    idx_all, rows0, rows1, outc0, outc1, sem0, sem1, semo0, semo1 = refs[2 + ntab:]

    nc = plsc.get_sparse_core_info().num_cores
    wid = lax.axis_index("s") * nc + lax.axis_index("c")
    seg0 = wid * seg_per_worker
    nch = seg_per_worker // G
    rows = (rows0, rows1)
    sems = (sem0, sem1)
    outcs = (outc0, outc1)
    semos = (semo0, semo1)

    pltpu.sync_copy(ctx_hbm.at[wid], idx_all)

    pltpu.async_copy(tabs[0].at[idx_all.at[0]], rows0, sem0)

    def do_step(c, t, half):
        par = (half + t) % 2
        pltpu.make_async_copy(tabs[t].at[idx_all.at[c]], rows[par], sems[par]).wait()
        if t + 1 < ntab:
            pltpu.async_copy(tabs[t + 1].at[idx_all.at[c]], rows[1 - par],
                             sems[1 - par])
        else:
            @pl.when(c + 1 < nch)
            def _():
                pltpu.async_copy(tabs[0].at[idx_all.at[c + 1]], rows[1 - par],
                                 sems[1 - par])
        sbase = seg0 + c * G
        if t == 0:
            @pl.when(c >= 2)
            def _():
                pltpu.make_async_copy(
                    outcs[half], out_hbm.at[:, pl.ds(sbase, G)], semos[half]).wait()

        rv = rows[par]
        oc = outcs[half]

        def seg_body(g, carry2):
            a0 = jnp.zeros((L,), jnp.float32)
            a1 = jnp.zeros((L,), jnp.float32)
            r0 = g * S
            for s in range(S):
                a0 = a0 + rv[r0 + s, pl.ds(0, L)]
                a1 = a1 + rv[r0 + s, pl.ds(L, L)]
            oc[t, g, pl.ds(0, L)] = a0
            oc[t, g, pl.ds(L, L)] = a1
            return carry2

        lax.fori_loop(0, G, seg_body, 0)
        if t == ntab - 1:
            pltpu.async_copy(oc, out_hbm.at[:, pl.ds(sbase, G)], semos[half])

    def pair_body(c2, carry):
        for half in range(2):
            c = 2 * c2 + half
            for t in range(ntab):
                do_step(c, t, half)
        return carry

    lax.fori_loop(0, nch // 2, pair_body, 0)
    for half in range(2):
        pltpu.make_async_copy(
            outcs[half], out_hbm.at[:, pl.ds(seg0, G)], semos[half]).wait()


def _sc_segment_sums(ctx, tables, nsegs, S, E):
    ntab = len(tables)
    info = plsc.get_sparse_core_info()
    nworkers = info.num_cores * info.num_subcores
    seg_per_worker = nsegs // nworkers
    G = 40
    nch = seg_per_worker // G
    assert seg_per_worker % G == 0 and G % 8 == 0 and nch % 2 == 0

    ctx3 = ctx.reshape(nworkers, nch, G * S)
    mesh = plsc.VectorSubcoreMesh(core_axis_name="c", subcore_axis_name="s")
    kfn = pl.kernel(
        functools.partial(_seg_sum_body, ntab, seg_per_worker, G, S, E),
        out_type=jax.ShapeDtypeStruct((ntab, nsegs, E), jnp.float32),
        mesh=mesh,
        scratch_types=[
            pltpu.VMEM((nch, G * S), jnp.int32),
            pltpu.VMEM((G * S, E), jnp.float32),
            pltpu.VMEM((G * S, E), jnp.float32),
            pltpu.VMEM((ntab, G, E), jnp.float32),
            pltpu.VMEM((ntab, G, E), jnp.float32),
            pltpu.SemaphoreType.DMA,
            pltpu.SemaphoreType.DMA,
            pltpu.SemaphoreType.DMA,
            pltpu.SemaphoreType.DMA,
        ],
        compiler_params=pltpu.CompilerParams(use_tc_tiling_on_sc=False),
    )
    return kfn(ctx3, *tables)


def _chain_body(E, sa1_ref, sa2_ref, scl_ref, o_ref):
    sa1 = sa1_ref[...]
    sa2 = sa2_ref[...]
    scl = scl_ref[...]
    q = sa1 * (1.0 / E)
    attn = jax.nn.softmax(sa1 * q, axis=-1)
    q = q + sa2 * attn
    attn = jax.nn.softmax(sa2 * q, axis=-1)
    o_ref[...] = scl * attn


def _tc_chain(sa1, sa2, scl):
    nsegs, E = sa1.shape
    BLK = 2048
    grid = (nsegs // BLK,)
    spec = pl.BlockSpec((BLK, E), lambda i: (i, 0))
    return pl.pallas_call(
        functools.partial(_chain_body, E),
        grid=grid,
        in_specs=[spec, spec, spec],
        out_specs=spec,
        out_shape=jax.ShapeDtypeStruct((nsegs, E), jnp.float32),
    )(sa1, sa2, scl)


def kernel(context, A_tables, C_last):
    B, M, S = context.shape
    E = A_tables.shape[-1]
    nsegs = B * M
    ctx = context.reshape(-1)
    tables = (A_tables[1], A_tables[2], C_last)
    sums = _sc_segment_sums(ctx, tables, nsegs, S, E)
    out = _tc_chain(sums[0], sums[1], sums[2])
    return out.reshape(B, M, E)


# speedup vs baseline: 42.6825x; 1.2615x over previous
import functools

import jax
import jax.numpy as jnp
from jax import lax
from jax.experimental import pallas as pl
from jax.experimental.pallas import tpu as pltpu
from jax.experimental.pallas import tpu_sc as plsc

L = 16

_GDN = lax.GatherDimensionNumbers(
    offset_dims=(), collapsed_slice_dims=(0,), start_index_map=(0,))


def _lane_shuffle(x, perm):
    return lax.gather(x, perm[:, None], _GDN, slice_sizes=(1,),
                      mode=lax.GatherScatterMode.PROMISE_IN_BOUNDS)


def _butterfly(x, op):
    lanes = lax.iota(jnp.int32, L)
    for k in (1, 2, 4, 8):
        x = op(x, _lane_shuffle(x, lax.bitwise_xor(lanes, k)))
    return x


def _encoder_body(ntab, seg_per_worker, G, S, E, *refs):
    ctx_hbm = refs[0]
    tabs = refs[1:1 + ntab]
    out_hbm = refs[1 + ntab]
    idx_all, rows0, rows1, outc0, outc1, ov0, ov1, sem0, sem1, semo0, semo1 = (
        refs[2 + ntab:])

    nc = plsc.get_sparse_core_info().num_cores
    wid = lax.axis_index("s") * nc + lax.axis_index("c")
    seg0 = wid * seg_per_worker
    nch = seg_per_worker // G
    rows = (rows0, rows1)
    sems = (sem0, sem1)
    outcs = (outc0, outc1)
    ovs = (ov0, ov1)
    semos = (semo0, semo1)
    inv_e = 1.0 / E

    pltpu.sync_copy(ctx_hbm.at[wid], idx_all)

    pltpu.async_copy(tabs[0].at[idx_all.at[0]], rows0, sem0)

    def do_step(c, t, half):
        par = (half + t) % 2
        pltpu.make_async_copy(tabs[t].at[idx_all.at[c]], rows[par], sems[par]).wait()
        if t + 1 < ntab:
            pltpu.async_copy(tabs[t + 1].at[idx_all.at[c]], rows[1 - par],
                             sems[1 - par])
        else:
            @pl.when(c + 1 < nch)
            def _():
                pltpu.async_copy(tabs[0].at[idx_all.at[c + 1]], rows[1 - par],
                                 sems[1 - par])
        sbase = seg0 + c * G
        rv = rows[par]
        oc = outcs[half]
        ov = ovs[half]

        if t < ntab - 1:
            def seg_body(g, carry2):
                a0 = jnp.zeros((L,), jnp.float32)
                a1 = jnp.zeros((L,), jnp.float32)
                r0 = g * S
                for s in range(S):
                    a0 = a0 + rv[r0 + s, pl.ds(0, L)]
                    a1 = a1 + rv[r0 + s, pl.ds(L, L)]
                oc[t, g, pl.ds(0, L)] = a0
                oc[t, g, pl.ds(L, L)] = a1
                return carry2

            lax.fori_loop(0, G, seg_body, 0)
        else:
            @pl.when(c >= 2)
            def _():
                pltpu.make_async_copy(
                    ov, out_hbm.at[pl.ds(sbase, G)], semos[half]).wait()

            def seg_body(g, carry2):
                a0 = jnp.zeros((L,), jnp.float32)
                a1 = jnp.zeros((L,), jnp.float32)
                r0 = g * S
                for s in range(S):
                    a0 = a0 + rv[r0 + s, pl.ds(0, L)]
                    a1 = a1 + rv[r0 + s, pl.ds(L, L)]
                s1a = oc[0, g, pl.ds(0, L)]
                s1b = oc[0, g, pl.ds(L, L)]
                s2a = oc[1, g, pl.ds(0, L)]
                s2b = oc[1, g, pl.ds(L, L)]
                qa = s1a * inv_e
                qb = s1b * inv_e
                za = s1a * qa
                zb = s1b * qb
                m = _butterfly(jnp.maximum(za, zb), jnp.maximum)
                ea = jnp.exp(za - m)
                eb = jnp.exp(zb - m)
                r = 1.0 / _butterfly(ea + eb, jnp.add)
                qa = qa + s2a * ea * r
                qb = qb + s2b * eb * r
                za = s2a * qa
                zb = s2b * qb
                m = _butterfly(jnp.maximum(za, zb), jnp.maximum)
                ea = jnp.exp(za - m)
                eb = jnp.exp(zb - m)
                r = 1.0 / _butterfly(ea + eb, jnp.add)
                ov[g, pl.ds(0, L)] = a0 * ea * r
                ov[g, pl.ds(L, L)] = a1 * eb * r
                return carry2

            lax.fori_loop(0, G, seg_body, 0)
            pltpu.async_copy(ov, out_hbm.at[pl.ds(sbase, G)], semos[half])

    def pair_body(c2, carry):
        for half in range(2):
            c = 2 * c2 + half
            for t in range(ntab):
                do_step(c, t, half)
        return carry

    lax.fori_loop(0, nch // 2, pair_body, 0)
    for half in range(2):
        pltpu.make_async_copy(
            ovs[half], out_hbm.at[pl.ds(seg0, G)], semos[half]).wait()


def _sc_encoder(ctx, tables, nsegs, S, E):
    ntab = len(tables)
    info = plsc.get_sparse_core_info()
    nworkers = info.num_cores * info.num_subcores
    seg_per_worker = nsegs // nworkers
    G = 40
    nch = seg_per_worker // G
    assert seg_per_worker % G == 0 and G % 8 == 0 and nch % 2 == 0

    ctx3 = ctx.reshape(nworkers, nch, G * S)
    mesh = plsc.VectorSubcoreMesh(core_axis_name="c", subcore_axis_name="s")
    kfn = pl.kernel(
        functools.partial(_encoder_body, ntab, seg_per_worker, G, S, E),
        out_type=jax.ShapeDtypeStruct((nsegs, E), jnp.float32),
        mesh=mesh,
        scratch_types=[
            pltpu.VMEM((nch, G * S), jnp.int32),
            pltpu.VMEM((G * S, E), jnp.float32),
            pltpu.VMEM((G * S, E), jnp.float32),
            pltpu.VMEM((ntab - 1, G, E), jnp.float32),
            pltpu.VMEM((ntab - 1, G, E), jnp.float32),
            pltpu.VMEM((G, E), jnp.float32),
            pltpu.VMEM((G, E), jnp.float32),
            pltpu.SemaphoreType.DMA,
            pltpu.SemaphoreType.DMA,
            pltpu.SemaphoreType.DMA,
            pltpu.SemaphoreType.DMA,
        ],
        compiler_params=pltpu.CompilerParams(use_tc_tiling_on_sc=False),
    )
    return kfn(ctx3, *tables)


def kernel(context, A_tables, C_last):
    B, M, S = context.shape
    E = A_tables.shape[-1]
    nsegs = B * M
    ctx = context.reshape(-1)
    tables = (A_tables[1], A_tables[2], C_last)
    out = _sc_encoder(ctx, tables, nsegs, S, E)
    return out.reshape(B, M, E)


# speedup vs baseline: 48.1121x; 1.1272x over previous
import functools

import jax
import jax.numpy as jnp
from jax import lax
from jax.experimental import pallas as pl
from jax.experimental.pallas import tpu as pltpu
from jax.experimental.pallas import tpu_sc as plsc

L = 16

_GDN = lax.GatherDimensionNumbers(
    offset_dims=(), collapsed_slice_dims=(0,), start_index_map=(0,))


def _lane_shuffle(x, perm):
    return lax.gather(x, perm[:, None], _GDN, slice_sizes=(1,),
                      mode=lax.GatherScatterMode.PROMISE_IN_BOUNDS)


def _butterfly(x, op):
    lanes = lax.iota(jnp.int32, L)
    for k in (1, 2, 4, 8):
        x = op(x, _lane_shuffle(x, lax.bitwise_xor(lanes, k)))
    return x


def _encoder_body(ntab, seg_per_worker, G, S, E, *refs):
    ctx_hbm = refs[0]
    tabs = refs[1:1 + ntab]
    out_hbm = refs[1 + ntab]
    idx_all, rows0, rows1, outc0, outc1, ov0, ov1, sem0, sem1, semo0, semo1 = (
        refs[2 + ntab:])

    nc = plsc.get_sparse_core_info().num_cores
    wid = lax.axis_index("s") * nc + lax.axis_index("c")
    nch = seg_per_worker // G
    brow0 = wid * nch
    rows = (rows0, rows1)
    sems = (sem0, sem1)
    outcs = (outc0, outc1)
    ovs = (ov0, ov1)
    semos = (semo0, semo1)
    inv_e = 1.0 / E

    pltpu.sync_copy(ctx_hbm.at[wid], idx_all)

    pltpu.async_copy(tabs[0].at[idx_all.at[0]], rows0, sem0)

    def do_step(c, t, half):
        par = (half + t) % 2
        pltpu.make_async_copy(tabs[t].at[idx_all.at[c]], rows[par], sems[par]).wait()
        if t + 1 < ntab:
            pltpu.async_copy(tabs[t + 1].at[idx_all.at[c]], rows[1 - par],
                             sems[1 - par])
        else:
            @pl.when(c + 1 < nch)
            def _():
                pltpu.async_copy(tabs[0].at[idx_all.at[c + 1]], rows[1 - par],
                                 sems[1 - par])
        bidx = brow0 + c
        rv = rows[par]
        oc = outcs[half]
        ov = ovs[half]

        if t < ntab - 1:
            def seg_body(g, carry2):
                a0 = jnp.zeros((L,), jnp.float32)
                a1 = jnp.zeros((L,), jnp.float32)
                r0 = g * S
                for s in range(S):
                    a0 = a0 + rv[r0 + s, pl.ds(0, L)]
                    a1 = a1 + rv[r0 + s, pl.ds(L, L)]
                oc[t, g, pl.ds(0, L)] = a0
                oc[t, g, pl.ds(L, L)] = a1
                return carry2

            lax.fori_loop(0, G, seg_body, 0)
        else:
            @pl.when(c >= 2)
            def _():
                pltpu.make_async_copy(
                    ov, out_hbm.at[bidx], semos[half]).wait()

            def seg_body(g, carry2):
                a0 = jnp.zeros((L,), jnp.float32)
                a1 = jnp.zeros((L,), jnp.float32)
                r0 = g * S
                for s in range(S):
                    a0 = a0 + rv[r0 + s, pl.ds(0, L)]
                    a1 = a1 + rv[r0 + s, pl.ds(L, L)]
                s1a = oc[0, g, pl.ds(0, L)]
                s1b = oc[0, g, pl.ds(L, L)]
                s2a = oc[1, g, pl.ds(0, L)]
                s2b = oc[1, g, pl.ds(L, L)]
                qa = s1a * inv_e
                qb = s1b * inv_e
                za = s1a * qa
                zb = s1b * qb
                m = _butterfly(jnp.maximum(za, zb), jnp.maximum)
                ea = jnp.exp(za - m)
                eb = jnp.exp(zb - m)
                r = 1.0 / _butterfly(ea + eb, jnp.add)
                qa = qa + s2a * ea * r
                qb = qb + s2b * eb * r
                za = s2a * qa
                zb = s2b * qb
                m = _butterfly(jnp.maximum(za, zb), jnp.maximum)
                ea = jnp.exp(za - m)
                eb = jnp.exp(zb - m)
                r = 1.0 / _butterfly(ea + eb, jnp.add)
                ov[g, pl.ds(0, L)] = a0 * ea * r
                ov[g, pl.ds(L, L)] = a1 * eb * r
                return carry2

            lax.fori_loop(0, G, seg_body, 0)
            pltpu.async_copy(ov, out_hbm.at[bidx], semos[half])

    def pair_body(c2, carry):
        for half in range(2):
            c = 2 * c2 + half
            for t in range(ntab):
                do_step(c, t, half)
        return carry

    lax.fori_loop(0, nch // 2, pair_body, 0)
    for half in range(2):
        pltpu.make_async_copy(
            ovs[half], out_hbm.at[brow0], semos[half]).wait()


def _sc_encoder(ctx, tables, B, M, S, E):
    ntab = len(tables)
    info = plsc.get_sparse_core_info()
    nworkers = info.num_cores * info.num_subcores
    seg_per_worker = (B * M) // nworkers
    G = M
    nch = seg_per_worker // G
    assert seg_per_worker % G == 0 and nch % 2 == 0 and (G * S) % 8 == 0

    ctx3 = ctx.reshape(nworkers, nch, G * S)
    mesh = plsc.VectorSubcoreMesh(core_axis_name="c", subcore_axis_name="s")
    kfn = pl.kernel(
        functools.partial(_encoder_body, ntab, seg_per_worker, G, S, E),
        out_type=jax.ShapeDtypeStruct((B, M, E), jnp.float32),
        mesh=mesh,
        scratch_types=[
            pltpu.VMEM((nch, G * S), jnp.int32),
            pltpu.VMEM((G * S, E), jnp.float32),
            pltpu.VMEM((G * S, E), jnp.float32),
            pltpu.VMEM((ntab - 1, G, E), jnp.float32),
            pltpu.VMEM((ntab - 1, G, E), jnp.float32),
            pltpu.VMEM((G, E), jnp.float32),
            pltpu.VMEM((G, E), jnp.float32),
            pltpu.SemaphoreType.DMA,
            pltpu.SemaphoreType.DMA,
            pltpu.SemaphoreType.DMA,
            pltpu.SemaphoreType.DMA,
        ],
        compiler_params=pltpu.CompilerParams(use_tc_tiling_on_sc=False),
    )
    return kfn(ctx3, *tables)


def kernel(context, A_tables, C_last):
    B, M, S = context.shape
    E = A_tables.shape[-1]
    ctx = context.reshape(-1)
    tables = (A_tables[1], A_tables[2], C_last)
    return _sc_encoder(ctx, tables, B, M, S, E)


# speedup vs baseline: 53.0624x; 1.1029x over previous
import functools

import jax
import jax.numpy as jnp
from jax import lax
from jax.experimental import pallas as pl
from jax.experimental.pallas import tpu as pltpu
from jax.experimental.pallas import tpu_sc as plsc

L = 16

_GDN = lax.GatherDimensionNumbers(
    offset_dims=(), collapsed_slice_dims=(0,), start_index_map=(0,))


def _lane_shuffle(x, perm):
    return lax.gather(x, perm[:, None], _GDN, slice_sizes=(1,),
                      mode=lax.GatherScatterMode.PROMISE_IN_BOUNDS)


def _butterfly(x, op):
    lanes = lax.iota(jnp.int32, L)
    for k in (1, 2, 4, 8):
        x = op(x, _lane_shuffle(x, lax.bitwise_xor(lanes, k)))
    return x


def _worker_id():
    nc = plsc.get_sparse_core_info().num_cores
    return lax.axis_index("s") * nc + lax.axis_index("c")


def _seg_sum(rv, r0, S):
    a0 = jnp.zeros((L,), jnp.float32)
    a1 = jnp.zeros((L,), jnp.float32)
    for s in range(S):
        a0 = a0 + rv[r0 + s, pl.ds(0, L)]
        a1 = a1 + rv[r0 + s, pl.ds(L, L)]
    return a0, a1


def _gather_body(nch, G, S, E, ctx_hbm, tab, sums_hbm,
                 idx_all, rows0, rows1, acc0, acc1, sem0, sem1, semo0, semo1):
    wid = _worker_id()
    rows = (rows0, rows1)
    sems = (sem0, sem1)
    accs = (acc0, acc1)
    semos = (semo0, semo1)

    pltpu.sync_copy(ctx_hbm.at[wid], idx_all)
    pltpu.async_copy(tab.at[idx_all.at[0]], rows0, sem0)

    def do_chunk(c, par):
        pltpu.make_async_copy(tab.at[idx_all.at[c]], rows[par], sems[par]).wait()

        @pl.when(c + 1 < nch)
        def _():
            pltpu.async_copy(tab.at[idx_all.at[c + 1]], rows[1 - par],
                             sems[1 - par])

        @pl.when(c >= 2)
        def _():
            pltpu.make_async_copy(
                accs[par], sums_hbm.at[wid, c], semos[par]).wait()

        rv = rows[par]
        acc = accs[par]

        def seg_body(g, carry):
            a0, a1 = _seg_sum(rv, g * S, S)
            acc[g, pl.ds(0, L)] = a0
            acc[g, pl.ds(L, L)] = a1
            return carry

        lax.fori_loop(0, G, seg_body, 0)
        pltpu.async_copy(acc, sums_hbm.at[wid, c], semos[par])

    def pair_body(c2, carry):
        do_chunk(2 * c2, 0)
        do_chunk(2 * c2 + 1, 1)
        return carry

    lax.fori_loop(0, nch // 2, pair_body, 0)
    for par in range(2):
        pltpu.make_async_copy(accs[par], sums_hbm.at[wid, 0], semos[par]).wait()


def _final_body(nch, G, S, E, ctx_hbm, tab, s1_hbm, s2_hbm, out_hbm,
                idx_all, rows0, rows1, s1b0, s1b1, s2b0, s2b1, ov0, ov1,
                sem0, sem1, sems0, sems1, semo0, semo1):
    wid = _worker_id()
    brow0 = wid * nch
    rows = (rows0, rows1)
    sems = (sem0, sem1)
    s1bs = (s1b0, s1b1)
    s2bs = (s2b0, s2b1)
    sems_s = (sems0, sems1)
    ovs = (ov0, ov1)
    semos = (semo0, semo1)
    inv_e = 1.0 / E

    pltpu.sync_copy(ctx_hbm.at[wid], idx_all)
    pltpu.async_copy(tab.at[idx_all.at[0]], rows0, sem0)
    pltpu.async_copy(s1_hbm.at[wid, 0], s1b0, sems0)
    pltpu.async_copy(s2_hbm.at[wid, 0], s2b0, sems0)

    def do_chunk(c, par):
        pltpu.make_async_copy(tab.at[idx_all.at[c]], rows[par], sems[par]).wait()
        pltpu.make_async_copy(s1_hbm.at[wid, c], s1bs[par], sems_s[par]).wait()
        pltpu.make_async_copy(s2_hbm.at[wid, c], s2bs[par], sems_s[par]).wait()

        @pl.when(c + 1 < nch)
        def _():
            pltpu.async_copy(tab.at[idx_all.at[c + 1]], rows[1 - par],
                             sems[1 - par])
            pltpu.async_copy(s1_hbm.at[wid, c + 1], s1bs[1 - par],
                             sems_s[1 - par])
            pltpu.async_copy(s2_hbm.at[wid, c + 1], s2bs[1 - par],
                             sems_s[1 - par])

        @pl.when(c >= 2)
        def _():
            pltpu.make_async_copy(
                ovs[par], out_hbm.at[brow0 + c], semos[par]).wait()

        rv = rows[par]
        s1b = s1bs[par]
        s2b = s2bs[par]
        ov = ovs[par]

        def seg_body(g, carry):
            a0, a1 = _seg_sum(rv, g * S, S)
            s1a = s1b[g, pl.ds(0, L)]
            s1c = s1b[g, pl.ds(L, L)]
            s2a = s2b[g, pl.ds(0, L)]
            s2c = s2b[g, pl.ds(L, L)]
            qa = s1a * inv_e
            qb = s1c * inv_e
            za = s1a * qa
            zb = s1c * qb
            m = _butterfly(jnp.maximum(za, zb), jnp.maximum)
            ea = jnp.exp(za - m)
            eb = jnp.exp(zb - m)
            r = 1.0 / _butterfly(ea + eb, jnp.add)
            qa = qa + s2a * ea * r
            qb = qb + s2c * eb * r
            za = s2a * qa
            zb = s2c * qb
            m = _butterfly(jnp.maximum(za, zb), jnp.maximum)
            ea = jnp.exp(za - m)
            eb = jnp.exp(zb - m)
            r = 1.0 / _butterfly(ea + eb, jnp.add)
            ov[g, pl.ds(0, L)] = a0 * ea * r
            ov[g, pl.ds(L, L)] = a1 * eb * r
            return carry

        lax.fori_loop(0, G, seg_body, 0)
        pltpu.async_copy(ov, out_hbm.at[brow0 + c], semos[par])

    def pair_body(c2, carry):
        do_chunk(2 * c2, 0)
        do_chunk(2 * c2 + 1, 1)
        return carry

    lax.fori_loop(0, nch // 2, pair_body, 0)
    for par in range(2):
        pltpu.make_async_copy(ovs[par], out_hbm.at[brow0], semos[par]).wait()


def _sc_encoder(ctx, t1, t2, t3, B, M, S, E):
    info = plsc.get_sparse_core_info()
    nworkers = info.num_cores * info.num_subcores
    seg_per_worker = (B * M) // nworkers
    G = M
    nch = seg_per_worker // G
    assert seg_per_worker % G == 0 and nch % 2 == 0 and (G * S) % 8 == 0

    ctx3 = ctx.reshape(nworkers, nch, G * S)
    mesh = plsc.VectorSubcoreMesh(core_axis_name="c", subcore_axis_name="s")
    params = pltpu.CompilerParams(use_tc_tiling_on_sc=False)

    gather_fn = pl.kernel(
        functools.partial(_gather_body, nch, G, S, E),
        out_type=jax.ShapeDtypeStruct((nworkers, nch, G, E), jnp.float32),
        mesh=mesh,
        scratch_types=[
            pltpu.VMEM((nch, G * S), jnp.int32),
            pltpu.VMEM((G * S, E), jnp.float32),
            pltpu.VMEM((G * S, E), jnp.float32),
            pltpu.VMEM((G, E), jnp.float32),
            pltpu.VMEM((G, E), jnp.float32),
            pltpu.SemaphoreType.DMA,
            pltpu.SemaphoreType.DMA,
            pltpu.SemaphoreType.DMA,
            pltpu.SemaphoreType.DMA,
        ],
        compiler_params=params,
    )
    sums1 = gather_fn(ctx3, t1)
    sums2 = gather_fn(ctx3, t2)

    final_fn = pl.kernel(
        functools.partial(_final_body, nch, G, S, E),
        out_type=jax.ShapeDtypeStruct((B, M, E), jnp.float32),
        mesh=mesh,
        scratch_types=[
            pltpu.VMEM((nch, G * S), jnp.int32),
            pltpu.VMEM((G * S, E), jnp.float32),
            pltpu.VMEM((G * S, E), jnp.float32),
            pltpu.VMEM((G, E), jnp.float32),
            pltpu.VMEM((G, E), jnp.float32),
            pltpu.VMEM((G, E), jnp.float32),
            pltpu.VMEM((G, E), jnp.float32),
            pltpu.VMEM((G, E), jnp.float32),
            pltpu.VMEM((G, E), jnp.float32),
            pltpu.SemaphoreType.DMA,
            pltpu.SemaphoreType.DMA,
            pltpu.SemaphoreType.DMA,
            pltpu.SemaphoreType.DMA,
            pltpu.SemaphoreType.DMA,
            pltpu.SemaphoreType.DMA,
        ],
        compiler_params=params,
    )
    return final_fn(ctx3, t3, sums1, sums2)


def kernel(context, A_tables, C_last):
    B, M, S = context.shape
    E = A_tables.shape[-1]
    ctx = context.reshape(-1)
    return _sc_encoder(ctx, A_tables[1], A_tables[2], C_last, B, M, S, E)
